# baseline (speedup 1.0000x reference)
import jax
import jax.numpy as jnp
from jax import lax
from jax.experimental import pallas as pl
from jax.experimental.pallas import tpu as pltpu
from jax.experimental.pallas import tpu_sc as plsc

_BATCH = 16384
_D = 64
_NC = 2
_NS = 16
_NW = _NC * _NS
_BPW = _BATCH // _NW
_L = 16


def _sc_body(uid_hbm, iid_hbm, ut_hbm, it_hbm, out_hbm,
             uidx_v, iidx_v, urows_v, irows_v, out_v, sem_u, sem_i):
    wid = lax.axis_index("s") * _NC + lax.axis_index("c")
    base = wid * _BPW

    pltpu.sync_copy(uid_hbm.at[pl.ds(base, _BPW)], uidx_v)
    pltpu.sync_copy(iid_hbm.at[pl.ds(base, _BPW)], iidx_v)
    cu = pltpu.async_copy(ut_hbm.at[uidx_v], urows_v, sem_u)
    ci = pltpu.async_copy(it_hbm.at[iidx_v], irows_v, sem_i)
    cu.wait()
    ci.wait()

    row_iota = lax.iota(jnp.int32, _L)

    def chunk_body(c, carry):
        rows = c * _L + row_iota
        acc = jnp.zeros((_L,), jnp.float32)
        for d in range(_D):
            cols = jnp.full((_L,), d, jnp.int32)
            uu = plsc.load_gather(urows_v, [rows, cols])
            ii = plsc.load_gather(irows_v, [rows, cols])
            acc = acc + uu * ii
        out_v[pl.ds(c * _L, _L)] = acc
        return carry

    lax.fori_loop(0, _BPW // _L, chunk_body, 0)
    pltpu.sync_copy(out_v, out_hbm.at[pl.ds(base, _BPW)])


@jax.jit
def kernel(user_ids, item_ids, user_table, item_table):
    mesh = plsc.VectorSubcoreMesh(core_axis_name="c", subcore_axis_name="s")
    run = pl.kernel(
        _sc_body,
        mesh=mesh,
        out_type=jax.ShapeDtypeStruct((_BATCH,), jnp.float32),
        scratch_types=[
            pltpu.VMEM((_BPW,), jnp.int32),
            pltpu.VMEM((_BPW,), jnp.int32),
            pltpu.VMEM((_BPW, _D), jnp.float32),
            pltpu.VMEM((_BPW, _D), jnp.float32),
            pltpu.VMEM((_BPW,), jnp.float32),
            pltpu.SemaphoreType.DMA,
            pltpu.SemaphoreType.DMA,
        ],
        compiler_params=pltpu.CompilerParams(
            needs_layout_passes=False, use_tc_tiling_on_sc=False),
    )
    return run(user_ids.astype(jnp.int32), item_ids.astype(jnp.int32),
               user_table, item_table)


# speedup vs baseline: 1.9480x; 1.9480x over previous
import jax
import jax.numpy as jnp
from jax import lax
from jax.experimental import pallas as pl
from jax.experimental.pallas import tpu as pltpu
from jax.experimental.pallas import tpu_sc as plsc

_BATCH = 16384
_D = 64
_NC = 2
_NS = 16
_NW = _NC * _NS
_BPW = _BATCH // _NW
_L = 16
_C = 16
_NCHUNK = _BPW // _C


def _sc_body(uid_hbm, iid_hbm, ut_hbm, it_hbm, out_hbm,
             uids_v, iids_v,
             ubuf_v, ibuf_v, out_v, sem_u, sem_i):
    wid = lax.axis_index("s") * _NC + lax.axis_index("c")
    base = wid * _BPW

    pltpu.sync_copy(uid_hbm.at[pl.ds(base, _BPW)], uids_v)
    pltpu.sync_copy(iid_hbm.at[pl.ds(base, _BPW)], iids_v)

    lane = lax.iota(jnp.int32, _L)

    def chunk_body(c, carry):
        b0 = c * _C
        uids = uids_v[pl.ds(b0, _C)]
        iids = iids_v[pl.ds(b0, _C)]
        utix = lax.shift_right_logical(uids, 3)
        itix = lax.shift_right_logical(iids, 3)
        for j in range(_C):
            pltpu.async_copy(ut_hbm.at[utix[j]], ubuf_v.at[j], sem_u)
            pltpu.async_copy(it_hbm.at[itix[j]], ibuf_v.at[j], sem_i)
        for j in range(_C):
            pltpu.make_async_copy(ut_hbm.at[utix[j]], ubuf_v.at[j],
                                  sem_u).wait()
            pltpu.make_async_copy(it_hbm.at[itix[j]], ibuf_v.at[j],
                                  sem_i).wait()
        usub = jnp.bitwise_and(uids, 7)
        isub = jnp.bitwise_and(iids, 7)
        acc = jnp.zeros((_L,), jnp.float32)
        for d in range(_D):
            dv = jnp.full((_L,), d, jnp.int32)
            uu = plsc.load_gather(ubuf_v, [lane, usub, dv])
            ii = plsc.load_gather(ibuf_v, [lane, isub, dv])
            acc = acc + uu * ii
        out_v[pl.ds(b0, _C)] = acc
        return carry

    lax.fori_loop(0, _NCHUNK, chunk_body, 0)
    pltpu.sync_copy(out_v, out_hbm.at[pl.ds(base, _BPW)])


@jax.jit
def kernel(user_ids, item_ids, user_table, item_table):
    nu = user_table.shape[0]
    ni = item_table.shape[0]
    ut3 = user_table.reshape(nu // 8, 8, _D)
    it3 = item_table.reshape(ni // 8, 8, _D)
    mesh = plsc.VectorSubcoreMesh(core_axis_name="c", subcore_axis_name="s")
    run = pl.kernel(
        _sc_body,
        mesh=mesh,
        out_type=jax.ShapeDtypeStruct((_BATCH,), jnp.float32),
        scratch_types=[
            pltpu.VMEM((_BPW,), jnp.int32),
            pltpu.VMEM((_BPW,), jnp.int32),
            pltpu.VMEM((_C, 8, _D), jnp.float32),
            pltpu.VMEM((_C, 8, _D), jnp.float32),
            pltpu.VMEM((_BPW,), jnp.float32),
            pltpu.SemaphoreType.DMA,
            pltpu.SemaphoreType.DMA,
        ],
        compiler_params=pltpu.CompilerParams(needs_layout_passes=False),
    )
    return run(user_ids.astype(jnp.int32), item_ids.astype(jnp.int32),
               ut3, it3)
